# baseline (speedup 1.0000x reference)
import dataclasses
import functools

import jax
import jax.numpy as jnp
from jax import lax
from jax.experimental import pallas as pl
from jax.experimental.pallas import tpu as pltpu
from jax.experimental.pallas import tpu_sc as plsc

N = 10000
E = 160000
IN_FEATS = 256
HID = 512
NCLS = 47

NT = 16
NSC = 2
K = 125
NI = (E // NT) // K
IB = 40
RPT = 624
RPT_LAST = N - (NT - 1) * RPT
RZ = 16
OC = 48

EPT = E // (NT * NSC)
DEG_IT = -(-EPT // 16)
EPT_PAD = DEG_IT * 16
DEG_TAIL = EPT - (DEG_IT - 1) * 16


def _mesh():
    return plsc.VectorSubcoreMesh(core_axis_name="c", subcore_axis_name="s",
                                  num_cores=NSC, num_subcores=NT)


def _sc_params():
    cp = pltpu.CompilerParams()
    if "needs_layout_passes" in pltpu.CompilerParams.__dataclass_fields__:
        cp = dataclasses.replace(cp, needs_layout_passes=False)
    return cp


def _sc_degrees(edge_pad):
    @functools.partial(
        pl.kernel,
        out_type=jax.ShapeDtypeStruct((2, NT * NSC, 1, N), jnp.float32),
        mesh=_mesh(),
        compiler_params=_sc_params(),
        scratch_types=[
            pltpu.VMEM((EPT_PAD,), jnp.int32),
            pltpu.VMEM((EPT_PAD,), jnp.int32),
            pltpu.VMEM((N,), jnp.float32),
            pltpu.VMEM((N,), jnp.float32),
        ],
    )
    def k(e_hbm, out_hbm, src_v, dst_v, od_v, id_v):
        c = lax.axis_index("c")
        s = lax.axis_index("s")
        wid = c * NT + s
        pltpu.sync_copy(e_hbm.at[0, wid, 0], src_v)
        pltpu.sync_copy(e_hbm.at[1, wid, 0], dst_v)

        zero = jnp.zeros((16,), jnp.float32)

        @pl.loop(0, N // 16)
        def _(i):
            od_v[pl.ds(i * 16, 16)] = zero
            id_v[pl.ds(i * 16, 16)] = zero

        ones = jnp.ones((16,), jnp.float32)
        lanes = lax.iota(jnp.int32, 16)

        @pl.loop(0, DEG_IT)
        def _(i):
            valid = lanes < jnp.where(i == DEG_IT - 1, DEG_TAIL, 16)
            plsc.addupdate_scatter(od_v, [src_v[pl.ds(i * 16, 16)]], ones,
                                   mask=valid)
            plsc.addupdate_scatter(id_v, [dst_v[pl.ds(i * 16, 16)]], ones,
                                   mask=valid)

        pltpu.sync_copy(od_v, out_hbm.at[0, wid, 0])
        pltpu.sync_copy(id_v, out_hbm.at[1, wid, 0])

    return k(edge_pad)


def _sc_aggregate(zc, edge4):
    C, _, W = zc.shape
    P = C // NSC

    @functools.partial(
        pl.kernel,
        out_type=jax.ShapeDtypeStruct((C, N, W), jnp.float32),
        mesh=_mesh(),
        compiler_params=_sc_params(),
        scratch_types=[
            pltpu.VMEM((IB, K), jnp.int32),
            pltpu.VMEM((IB, K), jnp.int32),
            pltpu.VMEM((K, W), jnp.float32),
            pltpu.VMEM((K, W), jnp.float32),
            pltpu.VMEM((RZ, W), jnp.float32),
            pltpu.VMEM_SHARED((N, W), jnp.float32),
            pltpu.SemaphoreType.DMA,
            pltpu.SemaphoreType.DMA,
            pltpu.SemaphoreType.DMA,
            pltpu.SemaphoreType.DMA,
        ],
    )
    def k(z_hbm, e_hbm, out_hbm, src_v, dst_v, rows0_v, rows1_v, zb_v, acc,
          gs0, gs1, ss0, ss1):
        c = lax.axis_index("c")
        s = lax.axis_index("s")

        zero = jnp.zeros((16,), jnp.float32)

        @pl.loop(0, RZ)
        def _(r):
            @pl.loop(0, W // 16)
            def _(j):
                zb_v[r, pl.ds(j * 16, 16)] = zero

        base = pl.multiple_of(s * RPT, 8)
        last = s == NT - 1

        for p in range(P):
            chunk = c * P + p

            @pl.loop(0, RPT // RZ)
            def _(r):
                pltpu.sync_copy(zb_v, acc.at[pl.ds(base + r * RZ, RZ)])

            @pl.when(last)
            def _():
                pltpu.sync_copy(zb_v, acc.at[pl.ds(base + RPT, RZ)])

            plsc.subcore_barrier()

            @pl.loop(0, NI // IB)
            def _(blk):
                pltpu.sync_copy(e_hbm.at[0, s, pl.ds(blk * IB, IB)], src_v)
                pltpu.sync_copy(e_hbm.at[1, s, pl.ds(blk * IB, IB)], dst_v)
                pltpu.async_copy(z_hbm.at[chunk].at[src_v.at[0]],
                                 rows0_v, gs0)

                @pl.loop(0, IB // 2)
                def _(ii):
                    i0 = ii * 2
                    pltpu.make_async_copy(z_hbm.at[chunk].at[src_v.at[i0]],
                                          rows0_v, gs0).wait()

                    @pl.when(ii > 0)
                    def _():
                        pltpu.make_async_copy(rows1_v,
                                              acc.at[dst_v.at[i0 - 1]],
                                              ss1).wait()

                    pltpu.async_copy(z_hbm.at[chunk].at[src_v.at[i0 + 1]],
                                     rows1_v, gs1)
                    pltpu.async_copy(rows0_v, acc.at[dst_v.at[i0]], ss0,
                                     add=True)
                    pltpu.make_async_copy(z_hbm.at[chunk].at[src_v.at[i0 + 1]],
                                          rows1_v, gs1).wait()
                    pltpu.async_copy(rows1_v, acc.at[dst_v.at[i0 + 1]], ss1,
                                     add=True)

                    @pl.when(ii < IB // 2 - 1)
                    def _():
                        pltpu.make_async_copy(rows0_v, acc.at[dst_v.at[i0]],
                                              ss0).wait()
                        pltpu.async_copy(
                            z_hbm.at[chunk].at[src_v.at[i0 + 2]],
                            rows0_v, gs0)

                pltpu.make_async_copy(rows0_v, acc.at[dst_v.at[IB - 2]],
                                      ss0).wait()
                pltpu.make_async_copy(rows1_v, acc.at[dst_v.at[IB - 1]],
                                      ss1).wait()

            plsc.subcore_barrier()

            @pl.loop(0, RPT // OC)
            def _(j):
                off = pl.multiple_of(base + j * OC, 8)
                pltpu.sync_copy(acc.at[pl.ds(off, OC)],
                                out_hbm.at[chunk, pl.ds(off, OC)])

            @pl.when(last)
            def _():
                off = pl.multiple_of(base + RPT, 8)
                pltpu.sync_copy(acc.at[pl.ds(off, RPT_LAST - RPT)],
                                out_hbm.at[chunk, pl.ds(off, RPT_LAST - RPT)])

            if p < P - 1:
                plsc.subcore_barrier()

    return k(zc, edge4)


def _sc_aggregate_split(z, edge_s):
    _, W = z.shape
    NI2 = edge_s.shape[2]

    @functools.partial(
        pl.kernel,
        out_type=jax.ShapeDtypeStruct((NSC, N, W), jnp.float32),
        mesh=_mesh(),
        compiler_params=_sc_params(),
        scratch_types=[
            pltpu.VMEM((NI2, K), jnp.int32),
            pltpu.VMEM((NI2, K), jnp.int32),
            pltpu.VMEM((K, W), jnp.float32),
            pltpu.VMEM((K, W), jnp.float32),
            pltpu.VMEM((RZ, W), jnp.float32),
            pltpu.VMEM_SHARED((N, W), jnp.float32),
            pltpu.SemaphoreType.DMA,
            pltpu.SemaphoreType.DMA,
            pltpu.SemaphoreType.DMA,
            pltpu.SemaphoreType.DMA,
        ],
    )
    def k(z_hbm, e_hbm, out_hbm, src_v, dst_v, rows0_v, rows1_v, zb_v, acc,
          gs0, gs1, ss0, ss1):
        c = lax.axis_index("c")
        s = lax.axis_index("s")
        wid = c * NT + s
        pltpu.sync_copy(e_hbm.at[0, wid], src_v)
        pltpu.sync_copy(e_hbm.at[1, wid], dst_v)

        zero = jnp.zeros((16,), jnp.float32)

        @pl.loop(0, RZ)
        def _(r):
            @pl.loop(0, W // 16)
            def _(j):
                zb_v[r, pl.ds(j * 16, 16)] = zero

        base = pl.multiple_of(s * RPT, 8)
        last = s == NT - 1

        @pl.loop(0, RPT // RZ)
        def _(r):
            pltpu.sync_copy(zb_v, acc.at[pl.ds(base + r * RZ, RZ)])

        @pl.when(last)
        def _():
            pltpu.sync_copy(zb_v, acc.at[pl.ds(base + RPT, RZ)])

        plsc.subcore_barrier()

        pltpu.async_copy(z_hbm.at[src_v.at[0]], rows0_v, gs0)

        @pl.loop(0, NI2 // 2)
        def _(ii):
            i0 = ii * 2
            pltpu.make_async_copy(z_hbm.at[src_v.at[i0]],
                                  rows0_v, gs0).wait()

            @pl.when(ii > 0)
            def _():
                pltpu.make_async_copy(rows1_v, acc.at[dst_v.at[i0 - 1]],
                                      ss1).wait()

            pltpu.async_copy(z_hbm.at[src_v.at[i0 + 1]], rows1_v, gs1)
            pltpu.async_copy(rows0_v, acc.at[dst_v.at[i0]], ss0, add=True)
            pltpu.make_async_copy(z_hbm.at[src_v.at[i0 + 1]],
                                  rows1_v, gs1).wait()
            pltpu.async_copy(rows1_v, acc.at[dst_v.at[i0 + 1]], ss1, add=True)

            @pl.when(ii < NI2 // 2 - 1)
            def _():
                pltpu.make_async_copy(rows0_v, acc.at[dst_v.at[i0]],
                                      ss0).wait()
                pltpu.async_copy(z_hbm.at[src_v.at[i0 + 2]], rows0_v, gs0)

        pltpu.make_async_copy(rows0_v, acc.at[dst_v.at[NI2 - 2]], ss0).wait()
        pltpu.make_async_copy(rows1_v, acc.at[dst_v.at[NI2 - 1]], ss1).wait()

        plsc.subcore_barrier()

        @pl.loop(0, RPT // OC)
        def _(j):
            off = pl.multiple_of(base + j * OC, 8)
            pltpu.sync_copy(acc.at[pl.ds(off, OC)],
                            out_hbm.at[c, pl.ds(off, OC)])

        @pl.when(last)
        def _():
            off = pl.multiple_of(base + RPT, 8)
            pltpu.sync_copy(acc.at[pl.ds(off, RPT_LAST - RPT)],
                            out_hbm.at[c, pl.ds(off, RPT_LAST - RPT)])

    return k(z, edge_s)


BM = 1000
G = N // BM


def _inv_body(part_ref, inv2_ref):
    p = part_ref[...]
    od = jnp.sum(p[0:NT * NSC], axis=0)
    idg = jnp.sum(p[NT * NSC:], axis=0)
    inv = jax.lax.rsqrt(jnp.maximum(jnp.stack([od, idg]), 1.0))
    inv2_ref[...] = inv.T


def _scale_body(feat_ref, inv2_ref, fs_ref):
    fsc = feat_ref[...] * inv2_ref[...][:, 0:1]
    fs_ref[0] = fsc[:, 0:128]
    fs_ref[1] = fsc[:, 128:256]


def _tc_prep(deg_part, feat):
    inv2 = pl.pallas_call(
        _inv_body,
        out_shape=jax.ShapeDtypeStruct((N, 2), jnp.float32),
    )(deg_part)
    fs = pl.pallas_call(
        _scale_body,
        grid=(G,),
        in_specs=[
            pl.BlockSpec((BM, IN_FEATS), lambda i: (i, 0)),
            pl.BlockSpec((BM, 2), lambda i: (i, 0)),
        ],
        out_specs=pl.BlockSpec((2, BM, 128), lambda i: (0, i, 0)),
        out_shape=jax.ShapeDtypeStruct((2, N, 128), jnp.float32),
    )(feat, inv2)
    return inv2, fs


def _make_layer_body(nin, has_w1, nout, w_out):
    def body(*refs):
        if has_w1:
            (a_ref, w1_ref, b_ref, g_ref, be_ref, inv2_ref, w2_ref, z_ref,
             t_scr, st_scr) = refs
        else:
            (a_ref, b_ref, g_ref, be_ref, inv2_ref, w2_ref, z_ref,
             t_scr, st_scr) = refs
            w1_ref = None
        ph = pl.program_id(0)
        i = pl.program_id(1)

        @pl.when(ph == 0)
        def _():
            a = jnp.concatenate([a_ref[ci] for ci in range(nin)], axis=1)
            if w1_ref is not None:
                a = jnp.dot(a.astype(jnp.bfloat16), w1_ref[...],
                            preferred_element_type=jnp.float32)
            t = a * inv2_ref[...][:, 1:2] + b_ref[...]
            t_scr[pl.ds(i * BM, BM), :] = t

            @pl.when(i == 0)
            def _():
                st_scr[...] = jnp.zeros_like(st_scr)

            st_scr[0, :] += jnp.sum(t, axis=0)
            st_scr[1, :] += jnp.sum(t * t, axis=0)

        @pl.when(ph == 1)
        def _():
            t = t_scr[pl.ds(i * BM, BM), :]
            mu = st_scr[0, :] * (1.0 / N)
            var = st_scr[1, :] * (1.0 / N) - mu * mu
            u = (t - mu) * jax.lax.rsqrt(var + 1e-5) * g_ref[...] + be_ref[...]
            u = jnp.maximum(u, 0.0)
            u = u * inv2_ref[...][:, 0:1]
            z = jnp.dot(u.astype(jnp.bfloat16), w2_ref[...],
                        preferred_element_type=jnp.float32)
            if nout == 1:
                z_ref[...] = z
            else:
                for ci in range(nout):
                    z_ref[ci] = z[:, ci * w_out:(ci + 1) * w_out]
    return body


def _tc_layer(a, w1, b, g, be, inv2, w2, nout, w_out):
    nin = a.shape[0]
    has_w1 = w1 is not None
    if nout == 1:
        out_spec = pl.BlockSpec((BM, w_out), lambda p, i: (i, 0))
        out_shape = jax.ShapeDtypeStruct((N, w_out), jnp.float32)
    else:
        out_spec = pl.BlockSpec((nout, BM, w_out), lambda p, i: (0, i, 0))
        out_shape = jax.ShapeDtypeStruct((nout, N, w_out), jnp.float32)
    in_specs = [pl.BlockSpec((nin, BM, 128), lambda p, i: (0, i, 0))]
    args = [a]
    if has_w1:
        in_specs.append(pl.BlockSpec(w1.shape, lambda p, i: (0, 0)))
        args.append(w1)
    in_specs += [
        pl.BlockSpec((HID,), lambda p, i: (0,)),
        pl.BlockSpec((HID,), lambda p, i: (0,)),
        pl.BlockSpec((HID,), lambda p, i: (0,)),
        pl.BlockSpec((BM, 2), lambda p, i: (i, 0)),
        pl.BlockSpec(w2.shape, lambda p, i: (0, 0)),
    ]
    args += [b, g, be, inv2, w2]
    return pl.pallas_call(
        _make_layer_body(nin, has_w1, nout, w_out),
        grid=(2, G),
        in_specs=in_specs,
        out_specs=out_spec,
        out_shape=out_shape,
        scratch_shapes=[pltpu.VMEM((N, HID), jnp.float32),
                        pltpu.VMEM((2, HID), jnp.float32)],
    )(*args)


def _fin_body(ac_ref, b_ref, inv2_ref, o_ref):
    a = ac_ref[0] + ac_ref[1]
    o_ref[...] = a[:, :NCLS] * inv2_ref[...][:, 1:2] + b_ref[...]


def _tc_fin(ac, b2, inv2):
    return pl.pallas_call(
        _fin_body,
        grid=(G,),
        in_specs=[
            pl.BlockSpec((2, BM, 128), lambda i: (0, i, 0)),
            pl.BlockSpec((NCLS,), lambda i: (0,)),
            pl.BlockSpec((BM, 2), lambda i: (i, 0)),
        ],
        out_specs=pl.BlockSpec((BM, NCLS), lambda i: (i, 0)),
        out_shape=jax.ShapeDtypeStruct((N, NCLS), jnp.float32),
    )(ac, b2, inv2)


def kernel(feat, edge_index, W0, b0, g0, be0, W1, b1, g1, be1, W2, b2):
    edge_index = edge_index.astype(jnp.int32)

    e32 = edge_index.reshape(2, NT * NSC, EPT)
    e32 = jnp.pad(e32, ((0, 0), (0, 0), (0, EPT_PAD - EPT)))
    e32 = e32.reshape(2, NT * NSC, 1, EPT_PAD)
    edge4 = edge_index.reshape(2, NT, NI, K)
    edge_s = edge_index.reshape(2, NT * NSC, NI // 2, K)

    deg_part = _sc_degrees(e32)

    inv2, fs = _tc_prep(deg_part.reshape(2 * NT * NSC, N), feat)

    ax = _sc_aggregate(fs, edge4)

    bf16 = jnp.bfloat16
    z1 = _tc_layer(ax, W0.astype(bf16), b0, g0, be0, inv2,
                   W1.astype(bf16), 4, 128)

    a1 = _sc_aggregate(z1, edge4)

    W2p = jnp.pad(W2, ((0, 0), (0, 128 - NCLS))).astype(bf16)
    z2 = _tc_layer(a1, None, b1, g1, be1, inv2, W2p, 1, 128)

    a2 = _sc_aggregate_split(z2, edge_s)

    return _tc_fin(a2, b2, inv2)


# speedup vs baseline: 1.0091x; 1.0091x over previous
import dataclasses
import functools

import jax
import jax.numpy as jnp
from jax import lax
from jax.experimental import pallas as pl
from jax.experimental.pallas import tpu as pltpu
from jax.experimental.pallas import tpu_sc as plsc

N = 10000
E = 160000
IN_FEATS = 256
HID = 512
NCLS = 47

NT = 16
NSC = 2
K = 125
NI = (E // NT) // K
IB = 40
RPT = 624
RPT_LAST = N - (NT - 1) * RPT
RZ = 16
OC = 48

EPT = E // (NT * NSC)
DEG_IT = -(-EPT // 16)
EPT_PAD = DEG_IT * 16
DEG_TAIL = EPT - (DEG_IT - 1) * 16


def _mesh():
    return plsc.VectorSubcoreMesh(core_axis_name="c", subcore_axis_name="s",
                                  num_cores=NSC, num_subcores=NT)


def _sc_params():
    cp = pltpu.CompilerParams()
    if "needs_layout_passes" in pltpu.CompilerParams.__dataclass_fields__:
        cp = dataclasses.replace(cp, needs_layout_passes=False)
    return cp


def _sc_degrees(edge_pad):
    @functools.partial(
        pl.kernel,
        out_type=jax.ShapeDtypeStruct((2, NT * NSC, 1, N), jnp.float32),
        mesh=_mesh(),
        compiler_params=_sc_params(),
        scratch_types=[
            pltpu.VMEM((EPT_PAD,), jnp.int32),
            pltpu.VMEM((EPT_PAD,), jnp.int32),
            pltpu.VMEM((N,), jnp.float32),
            pltpu.VMEM((N,), jnp.float32),
        ],
    )
    def k(e_hbm, out_hbm, src_v, dst_v, od_v, id_v):
        c = lax.axis_index("c")
        s = lax.axis_index("s")
        wid = c * NT + s
        pltpu.sync_copy(e_hbm.at[0, wid, 0], src_v)
        pltpu.sync_copy(e_hbm.at[1, wid, 0], dst_v)

        zero = jnp.zeros((16,), jnp.float32)

        @pl.loop(0, N // 16)
        def _(i):
            od_v[pl.ds(i * 16, 16)] = zero
            id_v[pl.ds(i * 16, 16)] = zero

        ones = jnp.ones((16,), jnp.float32)
        lanes = lax.iota(jnp.int32, 16)

        @pl.loop(0, DEG_IT)
        def _(i):
            valid = lanes < jnp.where(i == DEG_IT - 1, DEG_TAIL, 16)
            plsc.addupdate_scatter(od_v, [src_v[pl.ds(i * 16, 16)]], ones,
                                   mask=valid)
            plsc.addupdate_scatter(id_v, [dst_v[pl.ds(i * 16, 16)]], ones,
                                   mask=valid)

        pltpu.sync_copy(od_v, out_hbm.at[0, wid, 0])
        pltpu.sync_copy(id_v, out_hbm.at[1, wid, 0])

    return k(edge_pad)


def _sc_aggregate(zc, edge4):
    C, _, W = zc.shape
    P = C // NSC

    @functools.partial(
        pl.kernel,
        out_type=jax.ShapeDtypeStruct((C, N, W), jnp.float32),
        mesh=_mesh(),
        compiler_params=_sc_params(),
        scratch_types=[
            pltpu.VMEM((IB, K), jnp.int32),
            pltpu.VMEM((IB, K), jnp.int32),
            pltpu.VMEM((K, W), jnp.float32),
            pltpu.VMEM((K, W), jnp.float32),
            pltpu.VMEM((RZ, W), jnp.float32),
            pltpu.VMEM_SHARED((N, W), jnp.float32),
            pltpu.SemaphoreType.DMA,
            pltpu.SemaphoreType.DMA,
            pltpu.SemaphoreType.DMA,
            pltpu.SemaphoreType.DMA,
        ],
    )
    def k(z_hbm, e_hbm, out_hbm, src_v, dst_v, rows0_v, rows1_v, zb_v, acc,
          gs0, gs1, ss0, ss1):
        c = lax.axis_index("c")
        s = lax.axis_index("s")

        zero = jnp.zeros((16,), jnp.float32)

        @pl.loop(0, RZ)
        def _(r):
            @pl.loop(0, W // 16)
            def _(j):
                zb_v[r, pl.ds(j * 16, 16)] = zero

        base = pl.multiple_of(s * RPT, 8)
        last = s == NT - 1

        for p in range(P):
            chunk = c * P + p

            @pl.loop(0, RPT // RZ)
            def _(r):
                pltpu.sync_copy(zb_v, acc.at[pl.ds(base + r * RZ, RZ)])

            @pl.when(last)
            def _():
                pltpu.sync_copy(zb_v, acc.at[pl.ds(base + RPT, RZ)])

            plsc.subcore_barrier()

            @pl.loop(0, NI // IB)
            def _(blk):
                pltpu.sync_copy(e_hbm.at[0, s, pl.ds(blk * IB, IB)], src_v)
                pltpu.sync_copy(e_hbm.at[1, s, pl.ds(blk * IB, IB)], dst_v)
                pltpu.async_copy(z_hbm.at[chunk].at[src_v.at[0]],
                                 rows0_v, gs0)

                @pl.loop(0, IB // 2)
                def _(ii):
                    i0 = ii * 2
                    pltpu.make_async_copy(z_hbm.at[chunk].at[src_v.at[i0]],
                                          rows0_v, gs0).wait()

                    @pl.when(ii > 0)
                    def _():
                        pltpu.make_async_copy(rows1_v,
                                              acc.at[dst_v.at[i0 - 1]],
                                              ss1).wait()

                    pltpu.async_copy(z_hbm.at[chunk].at[src_v.at[i0 + 1]],
                                     rows1_v, gs1)
                    pltpu.async_copy(rows0_v, acc.at[dst_v.at[i0]], ss0,
                                     add=True)
                    pltpu.make_async_copy(z_hbm.at[chunk].at[src_v.at[i0 + 1]],
                                          rows1_v, gs1).wait()
                    pltpu.async_copy(rows1_v, acc.at[dst_v.at[i0 + 1]], ss1,
                                     add=True)

                    @pl.when(ii < IB // 2 - 1)
                    def _():
                        pltpu.make_async_copy(rows0_v, acc.at[dst_v.at[i0]],
                                              ss0).wait()
                        pltpu.async_copy(
                            z_hbm.at[chunk].at[src_v.at[i0 + 2]],
                            rows0_v, gs0)

                pltpu.make_async_copy(rows0_v, acc.at[dst_v.at[IB - 2]],
                                      ss0).wait()
                pltpu.make_async_copy(rows1_v, acc.at[dst_v.at[IB - 1]],
                                      ss1).wait()

            plsc.subcore_barrier()

            @pl.loop(0, RPT // OC)
            def _(j):
                off = pl.multiple_of(base + j * OC, 8)
                pltpu.sync_copy(acc.at[pl.ds(off, OC)],
                                out_hbm.at[chunk, pl.ds(off, OC)])

            @pl.when(last)
            def _():
                off = pl.multiple_of(base + RPT, 8)
                pltpu.sync_copy(acc.at[pl.ds(off, RPT_LAST - RPT)],
                                out_hbm.at[chunk, pl.ds(off, RPT_LAST - RPT)])

            if p < P - 1:
                plsc.subcore_barrier()

    return k(zc, edge4)


def _sc_aggregate_split(z, edge_s):
    _, W = z.shape
    NI2 = edge_s.shape[2]

    @functools.partial(
        pl.kernel,
        out_type=jax.ShapeDtypeStruct((NSC, N, W), jnp.float32),
        mesh=_mesh(),
        compiler_params=_sc_params(),
        scratch_types=[
            pltpu.VMEM((NI2, K), jnp.int32),
            pltpu.VMEM((NI2, K), jnp.int32),
            pltpu.VMEM((K, W), jnp.float32),
            pltpu.VMEM((K, W), jnp.float32),
            pltpu.VMEM((RZ, W), jnp.float32),
            pltpu.VMEM_SHARED((N, W), jnp.float32),
            pltpu.SemaphoreType.DMA,
            pltpu.SemaphoreType.DMA,
            pltpu.SemaphoreType.DMA,
            pltpu.SemaphoreType.DMA,
        ],
    )
    def k(z_hbm, e_hbm, out_hbm, src_v, dst_v, rows0_v, rows1_v, zb_v, acc,
          gs0, gs1, ss0, ss1):
        c = lax.axis_index("c")
        s = lax.axis_index("s")
        wid = c * NT + s
        pltpu.sync_copy(e_hbm.at[0, wid], src_v)
        pltpu.sync_copy(e_hbm.at[1, wid], dst_v)

        zero = jnp.zeros((16,), jnp.float32)

        @pl.loop(0, RZ)
        def _(r):
            @pl.loop(0, W // 16)
            def _(j):
                zb_v[r, pl.ds(j * 16, 16)] = zero

        base = pl.multiple_of(s * RPT, 8)
        last = s == NT - 1

        @pl.loop(0, RPT // RZ)
        def _(r):
            pltpu.sync_copy(zb_v, acc.at[pl.ds(base + r * RZ, RZ)])

        @pl.when(last)
        def _():
            pltpu.sync_copy(zb_v, acc.at[pl.ds(base + RPT, RZ)])

        plsc.subcore_barrier()

        pltpu.async_copy(z_hbm.at[src_v.at[0]], rows0_v, gs0)

        @pl.loop(0, NI2 // 2)
        def _(ii):
            i0 = ii * 2
            pltpu.make_async_copy(z_hbm.at[src_v.at[i0]],
                                  rows0_v, gs0).wait()

            @pl.when(ii > 0)
            def _():
                pltpu.make_async_copy(rows1_v, acc.at[dst_v.at[i0 - 1]],
                                      ss1).wait()

            pltpu.async_copy(z_hbm.at[src_v.at[i0 + 1]], rows1_v, gs1)
            pltpu.async_copy(rows0_v, acc.at[dst_v.at[i0]], ss0, add=True)
            pltpu.make_async_copy(z_hbm.at[src_v.at[i0 + 1]],
                                  rows1_v, gs1).wait()
            pltpu.async_copy(rows1_v, acc.at[dst_v.at[i0 + 1]], ss1, add=True)

            @pl.when(ii < NI2 // 2 - 1)
            def _():
                pltpu.make_async_copy(rows0_v, acc.at[dst_v.at[i0]],
                                      ss0).wait()
                pltpu.async_copy(z_hbm.at[src_v.at[i0 + 2]], rows0_v, gs0)

        pltpu.make_async_copy(rows0_v, acc.at[dst_v.at[NI2 - 2]], ss0).wait()
        pltpu.make_async_copy(rows1_v, acc.at[dst_v.at[NI2 - 1]], ss1).wait()

        plsc.subcore_barrier()

        @pl.loop(0, RPT // OC)
        def _(j):
            off = pl.multiple_of(base + j * OC, 8)
            pltpu.sync_copy(acc.at[pl.ds(off, OC)],
                            out_hbm.at[c, pl.ds(off, OC)])

        @pl.when(last)
        def _():
            off = pl.multiple_of(base + RPT, 8)
            pltpu.sync_copy(acc.at[pl.ds(off, RPT_LAST - RPT)],
                            out_hbm.at[c, pl.ds(off, RPT_LAST - RPT)])

    return k(z, edge_s)


BM = 1000
G = N // BM


def _inv_body(part_ref, inv2_ref):
    p = part_ref[...]
    od = jnp.sum(p[0:NT * NSC], axis=0)
    idg = jnp.sum(p[NT * NSC:], axis=0)
    inv = jax.lax.rsqrt(jnp.maximum(jnp.stack([od, idg]), 1.0))
    inv2_ref[...] = inv.T


def _scale_body(feat_ref, inv2_ref, fs_ref):
    fsc = feat_ref[...] * inv2_ref[...][:, 0:1]
    fs_ref[0] = fsc[:, 0:128]
    fs_ref[1] = fsc[:, 128:256]


def _tc_prep(deg_part, feat):
    inv2 = pl.pallas_call(
        _inv_body,
        out_shape=jax.ShapeDtypeStruct((N, 2), jnp.float32),
    )(deg_part)
    fs = pl.pallas_call(
        _scale_body,
        grid=(G,),
        in_specs=[
            pl.BlockSpec((BM, IN_FEATS), lambda i: (i, 0)),
            pl.BlockSpec((BM, 2), lambda i: (i, 0)),
        ],
        out_specs=pl.BlockSpec((2, BM, 128), lambda i: (0, i, 0)),
        out_shape=jax.ShapeDtypeStruct((2, N, 128), jnp.float32),
    )(feat, inv2)
    return inv2, fs


def _make_layer_body(nin, has_w1, nout, w_out):
    def body(*refs):
        if has_w1:
            (a_ref, w1_ref, b_ref, g_ref, be_ref, inv2_ref, w2_ref, z_ref,
             t_scr, st_scr) = refs
        else:
            (a_ref, b_ref, g_ref, be_ref, inv2_ref, w2_ref, z_ref,
             t_scr, st_scr) = refs
            w1_ref = None
        ph = pl.program_id(0)
        i = pl.program_id(1)

        @pl.when(ph == 0)
        def _():
            a = jnp.concatenate([a_ref[ci] for ci in range(nin)], axis=1)
            if w1_ref is not None:
                a = jnp.dot(a.astype(jnp.bfloat16), w1_ref[...],
                            preferred_element_type=jnp.float32)
            t = a * inv2_ref[...][:, 1:2] + b_ref[...]
            t_scr[pl.ds(i * BM, BM), :] = t

            @pl.when(i == 0)
            def _():
                st_scr[...] = jnp.zeros_like(st_scr)

            st_scr[0, :] += jnp.sum(t, axis=0)
            st_scr[1, :] += jnp.sum(t * t, axis=0)

        @pl.when(ph == 1)
        def _():
            t = t_scr[pl.ds(i * BM, BM), :]
            mu = st_scr[0, :] * (1.0 / N)
            var = st_scr[1, :] * (1.0 / N) - mu * mu
            u = (t - mu) * jax.lax.rsqrt(var + 1e-5) * g_ref[...] + be_ref[...]
            u = jnp.maximum(u, 0.0)
            u = u * inv2_ref[...][:, 0:1]
            z = jnp.dot(u.astype(jnp.bfloat16), w2_ref[...],
                        preferred_element_type=jnp.float32)
            if nout == 1:
                z_ref[...] = z
            else:
                for ci in range(nout):
                    z_ref[ci] = z[:, ci * w_out:(ci + 1) * w_out]
    return body


def _tc_layer(a, w1, b, g, be, inv2, w2, nout, w_out):
    nin = a.shape[0]
    has_w1 = w1 is not None
    if nout == 1:
        out_spec = pl.BlockSpec((BM, w_out), lambda p, i: (i, 0))
        out_shape = jax.ShapeDtypeStruct((N, w_out), jnp.float32)
    else:
        out_spec = pl.BlockSpec((nout, BM, w_out), lambda p, i: (0, i, 0))
        out_shape = jax.ShapeDtypeStruct((nout, N, w_out), jnp.float32)
    in_specs = [pl.BlockSpec((nin, BM, 128), lambda p, i: (0, i * (1 - p), 0))]
    args = [a]
    if has_w1:
        in_specs.append(pl.BlockSpec(w1.shape, lambda p, i: (0, 0)))
        args.append(w1)
    in_specs += [
        pl.BlockSpec((HID,), lambda p, i: (0,)),
        pl.BlockSpec((HID,), lambda p, i: (0,)),
        pl.BlockSpec((HID,), lambda p, i: (0,)),
        pl.BlockSpec((BM, 2), lambda p, i: (i, 0)),
        pl.BlockSpec(w2.shape, lambda p, i: (0, 0)),
    ]
    args += [b, g, be, inv2, w2]
    return pl.pallas_call(
        _make_layer_body(nin, has_w1, nout, w_out),
        grid=(2, G),
        in_specs=in_specs,
        out_specs=out_spec,
        out_shape=out_shape,
        scratch_shapes=[pltpu.VMEM((N, HID), jnp.float32),
                        pltpu.VMEM((2, HID), jnp.float32)],
    )(*args)


def _fin_body(ac_ref, b_ref, inv2_ref, o_ref):
    a = ac_ref[0] + ac_ref[1]
    o_ref[...] = a[:, :NCLS] * inv2_ref[...][:, 1:2] + b_ref[...]


def _tc_fin(ac, b2, inv2):
    return pl.pallas_call(
        _fin_body,
        grid=(G,),
        in_specs=[
            pl.BlockSpec((2, BM, 128), lambda i: (0, i, 0)),
            pl.BlockSpec((NCLS,), lambda i: (0,)),
            pl.BlockSpec((BM, 2), lambda i: (i, 0)),
        ],
        out_specs=pl.BlockSpec((BM, NCLS), lambda i: (i, 0)),
        out_shape=jax.ShapeDtypeStruct((N, NCLS), jnp.float32),
    )(ac, b2, inv2)


def kernel(feat, edge_index, W0, b0, g0, be0, W1, b1, g1, be1, W2, b2):
    edge_index = edge_index.astype(jnp.int32)

    e32 = edge_index.reshape(2, NT * NSC, EPT)
    e32 = jnp.pad(e32, ((0, 0), (0, 0), (0, EPT_PAD - EPT)))
    e32 = e32.reshape(2, NT * NSC, 1, EPT_PAD)
    edge4 = edge_index.reshape(2, NT, NI, K)
    edge_s = edge_index.reshape(2, NT * NSC, NI // 2, K)

    deg_part = _sc_degrees(e32)

    inv2, fs = _tc_prep(deg_part.reshape(2 * NT * NSC, N), feat)

    ax = _sc_aggregate(fs, edge4)

    bf16 = jnp.bfloat16
    z1 = _tc_layer(ax, W0.astype(bf16), b0, g0, be0, inv2,
                   W1.astype(bf16), 4, 128)

    a1 = _sc_aggregate(z1, edge4)

    W2p = jnp.pad(W2, ((0, 0), (0, 128 - NCLS))).astype(bf16)
    z2 = _tc_layer(a1, None, b1, g1, be1, inv2, W2p, 1, 128)

    a2 = _sc_aggregate_split(z2, edge_s)

    return _tc_fin(a2, b2, inv2)


# speedup vs baseline: 1.0350x; 1.0256x over previous
import dataclasses
import functools

import jax
import jax.numpy as jnp
from jax import lax
from jax.experimental import pallas as pl
from jax.experimental.pallas import tpu as pltpu
from jax.experimental.pallas import tpu_sc as plsc

N = 10000
E = 160000
IN_FEATS = 256
HID = 512
NCLS = 47

NT = 16
NSC = 2
K = 125
NI = (E // NT) // K
IB = 40
RPT = 624
RPT_LAST = N - (NT - 1) * RPT
RZ = 16
OC = 48

EPT = E // (NT * NSC)
DEG_IT = -(-EPT // 16)
EPT_PAD = DEG_IT * 16
DEG_TAIL = EPT - (DEG_IT - 1) * 16


def _mesh():
    return plsc.VectorSubcoreMesh(core_axis_name="c", subcore_axis_name="s",
                                  num_cores=NSC, num_subcores=NT)


def _sc_params():
    cp = pltpu.CompilerParams()
    if "needs_layout_passes" in pltpu.CompilerParams.__dataclass_fields__:
        cp = dataclasses.replace(cp, needs_layout_passes=False)
    return cp


def _sc_degrees(edge_pad):
    @functools.partial(
        pl.kernel,
        out_type=jax.ShapeDtypeStruct((2, NT * NSC, 1, N), jnp.float32),
        mesh=_mesh(),
        compiler_params=_sc_params(),
        scratch_types=[
            pltpu.VMEM((EPT_PAD,), jnp.int32),
            pltpu.VMEM((EPT_PAD,), jnp.int32),
            pltpu.VMEM((N,), jnp.float32),
            pltpu.VMEM((N,), jnp.float32),
        ],
    )
    def k(e_hbm, out_hbm, src_v, dst_v, od_v, id_v):
        c = lax.axis_index("c")
        s = lax.axis_index("s")
        wid = c * NT + s
        pltpu.sync_copy(e_hbm.at[0, wid, 0], src_v)
        pltpu.sync_copy(e_hbm.at[1, wid, 0], dst_v)

        zero = jnp.zeros((16,), jnp.float32)

        @pl.loop(0, N // 16)
        def _(i):
            od_v[pl.ds(i * 16, 16)] = zero
            id_v[pl.ds(i * 16, 16)] = zero

        ones = jnp.ones((16,), jnp.float32)
        lanes = lax.iota(jnp.int32, 16)

        @pl.loop(0, DEG_IT)
        def _(i):
            valid = lanes < jnp.where(i == DEG_IT - 1, DEG_TAIL, 16)
            plsc.addupdate_scatter(od_v, [src_v[pl.ds(i * 16, 16)]], ones,
                                   mask=valid)
            plsc.addupdate_scatter(id_v, [dst_v[pl.ds(i * 16, 16)]], ones,
                                   mask=valid)

        pltpu.sync_copy(od_v, out_hbm.at[0, wid, 0])
        pltpu.sync_copy(id_v, out_hbm.at[1, wid, 0])

    return k(edge_pad)


def _sc_aggregate(zc, edge4):
    C, _, W = zc.shape
    P = C // NSC

    @functools.partial(
        pl.kernel,
        out_type=jax.ShapeDtypeStruct((C, N, W), jnp.float32),
        mesh=_mesh(),
        compiler_params=_sc_params(),
        scratch_types=[
            pltpu.VMEM((IB, K), jnp.int32),
            pltpu.VMEM((IB, K), jnp.int32),
            pltpu.VMEM((K, W), jnp.float32),
            pltpu.VMEM((K, W), jnp.float32),
            pltpu.VMEM((RZ, W), jnp.float32),
            pltpu.VMEM_SHARED((N, W), jnp.float32),
            pltpu.SemaphoreType.DMA,
            pltpu.SemaphoreType.DMA,
            pltpu.SemaphoreType.DMA,
            pltpu.SemaphoreType.DMA,
        ],
    )
    def k(z_hbm, e_hbm, out_hbm, src_v, dst_v, rows0_v, rows1_v, zb_v, acc,
          gs0, gs1, ss0, ss1):
        c = lax.axis_index("c")
        s = lax.axis_index("s")

        zero = jnp.zeros((16,), jnp.float32)

        @pl.loop(0, RZ)
        def _(r):
            @pl.loop(0, W // 16)
            def _(j):
                zb_v[r, pl.ds(j * 16, 16)] = zero

        base = pl.multiple_of(s * RPT, 8)
        last = s == NT - 1

        for p in range(P):
            chunk = c * P + p

            @pl.loop(0, RPT // RZ)
            def _(r):
                pltpu.sync_copy(zb_v, acc.at[pl.ds(base + r * RZ, RZ)])

            @pl.when(last)
            def _():
                pltpu.sync_copy(zb_v, acc.at[pl.ds(base + RPT, RZ)])

            plsc.subcore_barrier()

            @pl.loop(0, NI // IB)
            def _(blk):
                pltpu.sync_copy(e_hbm.at[0, s, pl.ds(blk * IB, IB)], src_v)
                pltpu.sync_copy(e_hbm.at[1, s, pl.ds(blk * IB, IB)], dst_v)
                pltpu.async_copy(z_hbm.at[chunk].at[src_v.at[0]],
                                 rows0_v, gs0)

                @pl.loop(0, IB // 2)
                def _(ii):
                    i0 = ii * 2
                    pltpu.make_async_copy(z_hbm.at[chunk].at[src_v.at[i0]],
                                          rows0_v, gs0).wait()

                    @pl.when(ii > 0)
                    def _():
                        pltpu.make_async_copy(rows1_v,
                                              acc.at[dst_v.at[i0 - 1]],
                                              ss1).wait()

                    pltpu.async_copy(z_hbm.at[chunk].at[src_v.at[i0 + 1]],
                                     rows1_v, gs1)
                    pltpu.async_copy(rows0_v, acc.at[dst_v.at[i0]], ss0,
                                     add=True)
                    pltpu.make_async_copy(z_hbm.at[chunk].at[src_v.at[i0 + 1]],
                                          rows1_v, gs1).wait()
                    pltpu.async_copy(rows1_v, acc.at[dst_v.at[i0 + 1]], ss1,
                                     add=True)

                    @pl.when(ii < IB // 2 - 1)
                    def _():
                        pltpu.make_async_copy(rows0_v, acc.at[dst_v.at[i0]],
                                              ss0).wait()
                        pltpu.async_copy(
                            z_hbm.at[chunk].at[src_v.at[i0 + 2]],
                            rows0_v, gs0)

                pltpu.make_async_copy(rows0_v, acc.at[dst_v.at[IB - 2]],
                                      ss0).wait()
                pltpu.make_async_copy(rows1_v, acc.at[dst_v.at[IB - 1]],
                                      ss1).wait()

            plsc.subcore_barrier()

            @pl.loop(0, RPT // OC)
            def _(j):
                off = pl.multiple_of(base + j * OC, 8)
                pltpu.sync_copy(acc.at[pl.ds(off, OC)],
                                out_hbm.at[chunk, pl.ds(off, OC)])

            @pl.when(last)
            def _():
                off = pl.multiple_of(base + RPT, 8)
                pltpu.sync_copy(acc.at[pl.ds(off, RPT_LAST - RPT)],
                                out_hbm.at[chunk, pl.ds(off, RPT_LAST - RPT)])

            if p < P - 1:
                plsc.subcore_barrier()

    return k(zc, edge4)


def _sc_aggregate_split(z, edge_s):
    _, W = z.shape
    NI2 = edge_s.shape[2]

    @functools.partial(
        pl.kernel,
        out_type=jax.ShapeDtypeStruct((NSC, N, W), jnp.float32),
        mesh=_mesh(),
        compiler_params=_sc_params(),
        scratch_types=[
            pltpu.VMEM((NI2, K), jnp.int32),
            pltpu.VMEM((NI2, K), jnp.int32),
            pltpu.VMEM((K, W), jnp.float32),
            pltpu.VMEM((K, W), jnp.float32),
            pltpu.VMEM((RZ, W), jnp.float32),
            pltpu.VMEM_SHARED((N, W), jnp.float32),
            pltpu.SemaphoreType.DMA,
            pltpu.SemaphoreType.DMA,
            pltpu.SemaphoreType.DMA,
            pltpu.SemaphoreType.DMA,
        ],
    )
    def k(z_hbm, e_hbm, out_hbm, src_v, dst_v, rows0_v, rows1_v, zb_v, acc,
          gs0, gs1, ss0, ss1):
        c = lax.axis_index("c")
        s = lax.axis_index("s")
        wid = c * NT + s
        pltpu.sync_copy(e_hbm.at[0, wid], src_v)
        pltpu.sync_copy(e_hbm.at[1, wid], dst_v)

        zero = jnp.zeros((16,), jnp.float32)

        @pl.loop(0, RZ)
        def _(r):
            @pl.loop(0, W // 16)
            def _(j):
                zb_v[r, pl.ds(j * 16, 16)] = zero

        base = pl.multiple_of(s * RPT, 8)
        last = s == NT - 1

        @pl.loop(0, RPT // RZ)
        def _(r):
            pltpu.sync_copy(zb_v, acc.at[pl.ds(base + r * RZ, RZ)])

        @pl.when(last)
        def _():
            pltpu.sync_copy(zb_v, acc.at[pl.ds(base + RPT, RZ)])

        plsc.subcore_barrier()

        pltpu.async_copy(z_hbm.at[src_v.at[0]], rows0_v, gs0)

        @pl.loop(0, NI2 // 2)
        def _(ii):
            i0 = ii * 2
            pltpu.make_async_copy(z_hbm.at[src_v.at[i0]],
                                  rows0_v, gs0).wait()

            @pl.when(ii > 0)
            def _():
                pltpu.make_async_copy(rows1_v, acc.at[dst_v.at[i0 - 1]],
                                      ss1).wait()

            pltpu.async_copy(z_hbm.at[src_v.at[i0 + 1]], rows1_v, gs1)
            pltpu.async_copy(rows0_v, acc.at[dst_v.at[i0]], ss0, add=True)
            pltpu.make_async_copy(z_hbm.at[src_v.at[i0 + 1]],
                                  rows1_v, gs1).wait()
            pltpu.async_copy(rows1_v, acc.at[dst_v.at[i0 + 1]], ss1, add=True)

            @pl.when(ii < NI2 // 2 - 1)
            def _():
                pltpu.make_async_copy(rows0_v, acc.at[dst_v.at[i0]],
                                      ss0).wait()
                pltpu.async_copy(z_hbm.at[src_v.at[i0 + 2]], rows0_v, gs0)

        pltpu.make_async_copy(rows0_v, acc.at[dst_v.at[NI2 - 2]], ss0).wait()
        pltpu.make_async_copy(rows1_v, acc.at[dst_v.at[NI2 - 1]], ss1).wait()

        plsc.subcore_barrier()

        @pl.loop(0, RPT // OC)
        def _(j):
            off = pl.multiple_of(base + j * OC, 8)
            pltpu.sync_copy(acc.at[pl.ds(off, OC)],
                            out_hbm.at[c, pl.ds(off, OC)])

        @pl.when(last)
        def _():
            off = pl.multiple_of(base + RPT, 8)
            pltpu.sync_copy(acc.at[pl.ds(off, RPT_LAST - RPT)],
                            out_hbm.at[c, pl.ds(off, RPT_LAST - RPT)])

    return k(z, edge_s)


BM = 2000
G = N // BM


def _inv_body(part_ref, inv2_ref):
    p = part_ref[...]
    od = jnp.sum(p[0:NT * NSC], axis=0)
    idg = jnp.sum(p[NT * NSC:], axis=0)
    inv = jax.lax.rsqrt(jnp.maximum(jnp.stack([od, idg]), 1.0))
    inv2_ref[...] = inv.T


def _scale_body(feat_ref, inv2_ref, fs_ref):
    fsc = feat_ref[...] * inv2_ref[...][:, 0:1]
    fs_ref[0] = fsc[:, 0:128]
    fs_ref[1] = fsc[:, 128:256]


def _tc_prep(deg_part, feat):
    inv2 = pl.pallas_call(
        _inv_body,
        out_shape=jax.ShapeDtypeStruct((N, 2), jnp.float32),
    )(deg_part)
    fs = pl.pallas_call(
        _scale_body,
        grid=(G,),
        in_specs=[
            pl.BlockSpec((BM, IN_FEATS), lambda i: (i, 0)),
            pl.BlockSpec((BM, 2), lambda i: (i, 0)),
        ],
        out_specs=pl.BlockSpec((2, BM, 128), lambda i: (0, i, 0)),
        out_shape=jax.ShapeDtypeStruct((2, N, 128), jnp.float32),
    )(feat, inv2)
    return inv2, fs


def _make_layer_body(nin, has_w1, nout, w_out):
    def body(*refs):
        if has_w1:
            (a_ref, w1_ref, b_ref, g_ref, be_ref, inv2_ref, w2_ref, z_ref,
             t_scr, st_scr) = refs
        else:
            (a_ref, b_ref, g_ref, be_ref, inv2_ref, w2_ref, z_ref,
             t_scr, st_scr) = refs
            w1_ref = None
        ph = pl.program_id(0)
        i = pl.program_id(1)

        @pl.when(ph == 0)
        def _():
            a = jnp.concatenate([a_ref[ci] for ci in range(nin)], axis=1)
            if w1_ref is not None:
                a = jnp.dot(a.astype(jnp.bfloat16), w1_ref[...],
                            preferred_element_type=jnp.float32)
            t = a * inv2_ref[...][:, 1:2] + b_ref[...]
            t_scr[pl.ds(i * BM, BM), :] = t

            @pl.when(i == 0)
            def _():
                st_scr[...] = jnp.zeros_like(st_scr)

            st_scr[0, :] += jnp.sum(t, axis=0)
            st_scr[1, :] += jnp.sum(t * t, axis=0)

        @pl.when(ph == 1)
        def _():
            t = t_scr[pl.ds(i * BM, BM), :]
            mu = st_scr[0, :] * (1.0 / N)
            var = st_scr[1, :] * (1.0 / N) - mu * mu
            u = (t - mu) * jax.lax.rsqrt(var + 1e-5) * g_ref[...] + be_ref[...]
            u = jnp.maximum(u, 0.0)
            u = u * inv2_ref[...][:, 0:1]
            z = jnp.dot(u.astype(jnp.bfloat16), w2_ref[...],
                        preferred_element_type=jnp.float32)
            if nout == 1:
                z_ref[...] = z
            else:
                for ci in range(nout):
                    z_ref[ci] = z[:, ci * w_out:(ci + 1) * w_out]
    return body


def _tc_layer(a, w1, b, g, be, inv2, w2, nout, w_out):
    nin = a.shape[0]
    has_w1 = w1 is not None
    if nout == 1:
        out_spec = pl.BlockSpec((BM, w_out), lambda p, i: (i, 0))
        out_shape = jax.ShapeDtypeStruct((N, w_out), jnp.float32)
    else:
        out_spec = pl.BlockSpec((nout, BM, w_out), lambda p, i: (0, i, 0))
        out_shape = jax.ShapeDtypeStruct((nout, N, w_out), jnp.float32)
    in_specs = [pl.BlockSpec((nin, BM, 128), lambda p, i: (0, i * (1 - p), 0))]
    args = [a]
    if has_w1:
        in_specs.append(pl.BlockSpec(w1.shape, lambda p, i: (0, 0)))
        args.append(w1)
    in_specs += [
        pl.BlockSpec((HID,), lambda p, i: (0,)),
        pl.BlockSpec((HID,), lambda p, i: (0,)),
        pl.BlockSpec((HID,), lambda p, i: (0,)),
        pl.BlockSpec((BM, 2), lambda p, i: (i, 0)),
        pl.BlockSpec(w2.shape, lambda p, i: (0, 0)),
    ]
    args += [b, g, be, inv2, w2]
    return pl.pallas_call(
        _make_layer_body(nin, has_w1, nout, w_out),
        grid=(2, G),
        in_specs=in_specs,
        out_specs=out_spec,
        out_shape=out_shape,
        scratch_shapes=[pltpu.VMEM((N, HID), jnp.float32),
                        pltpu.VMEM((2, HID), jnp.float32)],
    )(*args)


def _fin_body(ac_ref, b_ref, inv2_ref, o_ref):
    a = ac_ref[0] + ac_ref[1]
    o_ref[...] = a[:, :NCLS] * inv2_ref[...][:, 1:2] + b_ref[...]


def _tc_fin(ac, b2, inv2):
    return pl.pallas_call(
        _fin_body,
        grid=(G,),
        in_specs=[
            pl.BlockSpec((2, BM, 128), lambda i: (0, i, 0)),
            pl.BlockSpec((NCLS,), lambda i: (0,)),
            pl.BlockSpec((BM, 2), lambda i: (i, 0)),
        ],
        out_specs=pl.BlockSpec((BM, NCLS), lambda i: (i, 0)),
        out_shape=jax.ShapeDtypeStruct((N, NCLS), jnp.float32),
    )(ac, b2, inv2)


def kernel(feat, edge_index, W0, b0, g0, be0, W1, b1, g1, be1, W2, b2):
    edge_index = edge_index.astype(jnp.int32)

    e32 = edge_index.reshape(2, NT * NSC, EPT)
    e32 = jnp.pad(e32, ((0, 0), (0, 0), (0, EPT_PAD - EPT)))
    e32 = e32.reshape(2, NT * NSC, 1, EPT_PAD)
    edge4 = edge_index.reshape(2, NT, NI, K)
    edge_s = edge_index.reshape(2, NT * NSC, NI // 2, K)

    deg_part = _sc_degrees(e32)

    inv2, fs = _tc_prep(deg_part.reshape(2 * NT * NSC, N), feat)

    ax = _sc_aggregate(fs, edge4)

    bf16 = jnp.bfloat16
    z1 = _tc_layer(ax, W0.astype(bf16), b0, g0, be0, inv2,
                   W1.astype(bf16), 4, 128)

    a1 = _sc_aggregate(z1, edge4)

    W2p = jnp.pad(W2, ((0, 0), (0, 128 - NCLS))).astype(bf16)
    z2 = _tc_layer(a1, None, b1, g1, be1, inv2, W2p, 1, 128)

    a2 = _sc_aggregate_split(z2, edge_s)

    return _tc_fin(a2, b2, inv2)
